# baseline (device time: 77698 ns/iter reference)
import jax
import jax.numpy as jnp
from jax import lax
from jax.experimental import pallas as pl
from jax.experimental.pallas import tpu as pltpu

N = 16
B = 2
SQ = 512
HL = 8
DH = 64
D_MODEL = 768
D_LOC = HL * DH
ROWS = B * SQ

RS_STAGES = [(1, 0), (4, 2), (2, 1), (8, 3)]
HALves = [ROWS >> (k + 1) for k in range(4)]


def _body(x_ref, wq_ref, k_ref, v_ref, wo_ref, out_ref,
          acc_ref, agb_ref, sb0, sb1, sb2, sb3, rx0, rx1, rx2, rx3,
          rs_ssem, rs_rsem, ag_ssem, ag_rsem):
    me = lax.axis_index("i")
    sb = [sb0, sb1, sb2, sb3]
    rx = [rx0, rx1, rx2, rx3]

    qb = lax.broadcasted_iota(jnp.int32, (SQ, SQ), 0) // 64
    kb = lax.broadcasted_iota(jnp.int32, (SQ, SQ), 1) // 64
    mask = kb <= qb
    bf16 = jnp.bfloat16
    wq_b = wq_ref[:, :].astype(bf16)
    wo_b = wo_ref[:, :].astype(bf16)
    for b in range(B):
        q = jnp.dot(x_ref[b].astype(bf16), wq_b,
                    preferred_element_type=jnp.float32)
        ctx_parts = []
        for h in range(HL):
            qh = q[:, h * DH:(h + 1) * DH].astype(bf16)
            kh = k_ref[b, :, h, :].astype(bf16)
            s = jnp.dot(qh, kh.T,
                        preferred_element_type=jnp.float32) * 0.125
            s = jnp.where(mask, s, -1e9)
            m = jnp.max(s, axis=-1, keepdims=True)
            w = jnp.exp(s - m)
            w = (w / jnp.sum(w, axis=-1, keepdims=True)).astype(bf16)
            ctx_parts.append(jnp.dot(w, v_ref[b, :, h, :].astype(bf16),
                                     preferred_element_type=jnp.float32))
        ctx = jnp.concatenate(ctx_parts, axis=1).astype(bf16)
        acc_ref[b * SQ:(b + 1) * SQ, :] = jnp.dot(
            ctx, wo_b, preferred_element_type=jnp.float32)

    barrier = pltpu.get_barrier_semaphore()
    for m, _ in RS_STAGES:
        pl.semaphore_signal(barrier, inc=1, device_id=(me ^ m,),
                            device_id_type=pl.DeviceIdType.MESH)
    pl.semaphore_wait(barrier, 4)

    base = jnp.int32(0)
    for k, (m, shift) in enumerate(RS_STAGES):
        half = HALves[k]
        bit = (me >> shift) & 1
        send_off = pl.multiple_of(base + (1 - bit) * half, 64)
        sb[k][...] = acc_ref[pl.ds(send_off, half)].astype(jnp.bfloat16)
        rdma = pltpu.make_async_remote_copy(
            src_ref=sb[k],
            dst_ref=rx[k],
            send_sem=rs_ssem.at[k],
            recv_sem=rs_rsem.at[k],
            device_id=(me ^ m,),
            device_id_type=pl.DeviceIdType.MESH,
        )
        rdma.start()
        rdma.wait()
        base = pl.multiple_of(base + bit * half, 64)
        acc_ref[pl.ds(base, half)] = (
            acc_ref[pl.ds(base, half)] + rx[k][...].astype(jnp.float32))

    agb_ref[pl.ds(base, HALves[3])] = (
        acc_ref[pl.ds(base, HALves[3])].astype(jnp.bfloat16))
    for k, (m, shift) in enumerate(reversed(RS_STAGES)):
        seg = HALves[3 - k]
        rdma = pltpu.make_async_remote_copy(
            src_ref=agb_ref.at[pl.ds(base, seg)],
            dst_ref=agb_ref.at[pl.ds(base, seg)],
            send_sem=ag_ssem.at[k],
            recv_sem=ag_rsem.at[k],
            device_id=(me ^ m,),
            device_id_type=pl.DeviceIdType.MESH,
        )
        rdma.start()
        rdma.wait()
        base = pl.multiple_of(base - (base & seg), 64)

    out_ref[0] = agb_ref[0:SQ, :].astype(jnp.float32)
    out_ref[1] = agb_ref[SQ:ROWS, :].astype(jnp.float32)


def kernel(x, Wq, K_ext, V_ext, Wo):
    me = lax.axis_index("i")
    wq_loc = lax.dynamic_slice(Wq, (0, me * D_LOC), (Wq.shape[0], D_LOC))
    wo_loc = lax.dynamic_slice(Wo, (me * D_LOC, 0), (D_LOC, Wo.shape[1]))

    return pl.pallas_call(
        _body,
        out_shape=jax.ShapeDtypeStruct((B, SQ, D_MODEL), jnp.float32),
        in_specs=[pl.BlockSpec(memory_space=pltpu.VMEM)] * 5,
        out_specs=pl.BlockSpec(memory_space=pltpu.VMEM),
        scratch_shapes=[
            pltpu.VMEM((ROWS, D_MODEL), jnp.float32),
            pltpu.VMEM((ROWS, D_MODEL), jnp.bfloat16),
            pltpu.VMEM((HALves[0], D_MODEL), jnp.bfloat16),
            pltpu.VMEM((HALves[1], D_MODEL), jnp.bfloat16),
            pltpu.VMEM((HALves[2], D_MODEL), jnp.bfloat16),
            pltpu.VMEM((HALves[3], D_MODEL), jnp.bfloat16),
            pltpu.VMEM((HALves[0], D_MODEL), jnp.bfloat16),
            pltpu.VMEM((HALves[1], D_MODEL), jnp.bfloat16),
            pltpu.VMEM((HALves[2], D_MODEL), jnp.bfloat16),
            pltpu.VMEM((HALves[3], D_MODEL), jnp.bfloat16),
            pltpu.SemaphoreType.DMA((4,)),
            pltpu.SemaphoreType.DMA((4,)),
            pltpu.SemaphoreType.DMA((4,)),
            pltpu.SemaphoreType.DMA((4,)),
        ],
        compiler_params=pltpu.CompilerParams(collective_id=0),
    )(x, wq_loc, K_ext, V_ext, wo_loc)


# device time: 61144 ns/iter; 1.2707x vs baseline; 1.2707x over previous
import jax
import jax.numpy as jnp
from jax import lax
from jax.experimental import pallas as pl
from jax.experimental.pallas import tpu as pltpu

N = 16
B = 2
SQ = 512
HL = 8
DH = 64
D_MODEL = 768
D_LOC = HL * DH
ROWS = B * SQ
NSCHED = 2
CW = D_MODEL // NSCHED

SCHED_STAGES = [
    [(1, 0), (4, 2), (2, 1), (8, 3)],
    [(4, 2), (1, 0), (8, 3), (2, 1)],
]
HALVES = [ROWS >> (k + 1) for k in range(4)]


def _body(x_ref, wq_ref, k_ref, v_ref, wo_ref, out_ref,
          acc_ref, agb_ref,
          sb0, sb1, sb2, sb3, sb4, sb5, sb6, sb7,
          rx0, rx1, rx2, rx3, rx4, rx5, rx6, rx7,
          rs_ssem, rs_rsem, ag_ssem, ag_rsem):
    me = lax.axis_index("i")
    sb = [[sb0, sb1, sb2, sb3], [sb4, sb5, sb6, sb7]]
    rx = [[rx0, rx1, rx2, rx3], [rx4, rx5, rx6, rx7]]

    qb = lax.broadcasted_iota(jnp.int32, (SQ, SQ), 0) // 64
    kb = lax.broadcasted_iota(jnp.int32, (SQ, SQ), 1) // 64
    mask = kb <= qb
    for b in range(B):
        q = jnp.dot(x_ref[b], wq_ref[:, :],
                    preferred_element_type=jnp.float32)
        ctx_parts = []
        for h in range(HL):
            qh = q[:, h * DH:(h + 1) * DH]
            kh = k_ref[b, :, h, :]
            s = jnp.dot(qh, kh.T,
                        preferred_element_type=jnp.float32) * 0.125
            s = jnp.where(mask, s, -1e9)
            m = jnp.max(s, axis=-1, keepdims=True)
            w = jnp.exp(s - m)
            w = w / jnp.sum(w, axis=-1, keepdims=True)
            ctx_parts.append(jnp.dot(w, v_ref[b, :, h, :],
                                     preferred_element_type=jnp.float32))
        ctx = jnp.concatenate(ctx_parts, axis=1)
        acc_ref[b * SQ:(b + 1) * SQ, :] = jnp.dot(
            ctx, wo_ref[:, :], preferred_element_type=jnp.float32)

    barrier = pltpu.get_barrier_semaphore()
    for m in (1, 2, 4, 8):
        pl.semaphore_signal(barrier, inc=1, device_id=(me ^ m,),
                            device_id_type=pl.DeviceIdType.MESH)
    pl.semaphore_wait(barrier, 4)

    base = [jnp.int32(0), jnp.int32(0)]
    for k in range(4):
        half = HALVES[k]
        rdmas = []
        for s in range(NSCHED):
            m, shift = SCHED_STAGES[s][k]
            bit = (me >> shift) & 1
            send_off = pl.multiple_of(base[s] + (1 - bit) * half, 64)
            sb[s][k][...] = acc_ref[
                pl.ds(send_off, half), s * CW:(s + 1) * CW].astype(jnp.bfloat16)
            rdma = pltpu.make_async_remote_copy(
                src_ref=sb[s][k],
                dst_ref=rx[s][k],
                send_sem=rs_ssem.at[s, k],
                recv_sem=rs_rsem.at[s, k],
                device_id=(me ^ m,),
                device_id_type=pl.DeviceIdType.MESH,
            )
            rdma.start()
            base[s] = pl.multiple_of(base[s] + bit * half, 64)
            rdmas.append(rdma)
        for s in range(NSCHED):
            rdmas[s].wait()
            acc_ref[pl.ds(base[s], half), s * CW:(s + 1) * CW] = (
                acc_ref[pl.ds(base[s], half), s * CW:(s + 1) * CW]
                + rx[s][k][...].astype(jnp.float32))

    for s in range(NSCHED):
        agb_ref[pl.ds(base[s], HALVES[3]), s * CW:(s + 1) * CW] = acc_ref[
            pl.ds(base[s], HALVES[3]), s * CW:(s + 1) * CW].astype(jnp.bfloat16)
    for k in range(4):
        seg = HALVES[3 - k]
        rdmas = []
        for s in range(NSCHED):
            m, _ = SCHED_STAGES[s][3 - k]
            rdma = pltpu.make_async_remote_copy(
                src_ref=agb_ref.at[pl.ds(base[s], seg), s * CW:(s + 1) * CW],
                dst_ref=agb_ref.at[pl.ds(base[s], seg), s * CW:(s + 1) * CW],
                send_sem=ag_ssem.at[s, k],
                recv_sem=ag_rsem.at[s, k],
                device_id=(me ^ m,),
                device_id_type=pl.DeviceIdType.MESH,
            )
            rdma.start()
            rdmas.append(rdma)
        for s in range(NSCHED):
            rdmas[s].wait()
            base[s] = pl.multiple_of(base[s] - (base[s] & seg), 64)

    out_ref[0] = agb_ref[0:SQ, :].astype(jnp.float32)
    out_ref[1] = agb_ref[SQ:ROWS, :].astype(jnp.float32)


def kernel(x, Wq, K_ext, V_ext, Wo):
    me = lax.axis_index("i")
    wq_loc = lax.dynamic_slice(Wq, (0, me * D_LOC), (Wq.shape[0], D_LOC))
    wo_loc = lax.dynamic_slice(Wo, (me * D_LOC, 0), (D_LOC, Wo.shape[1]))

    stage_bufs = [
        pltpu.VMEM((HALVES[k], CW), jnp.bfloat16)
        for _ in range(NSCHED) for k in range(4)
    ]
    return pl.pallas_call(
        _body,
        out_shape=jax.ShapeDtypeStruct((B, SQ, D_MODEL), jnp.float32),
        in_specs=[pl.BlockSpec(memory_space=pltpu.VMEM)] * 5,
        out_specs=pl.BlockSpec(memory_space=pltpu.VMEM),
        scratch_shapes=(
            [
                pltpu.VMEM((ROWS, D_MODEL), jnp.float32),
                pltpu.VMEM((ROWS, D_MODEL), jnp.bfloat16),
            ]
            + stage_bufs
            + stage_bufs
            + [
                pltpu.SemaphoreType.DMA((NSCHED, 4)),
                pltpu.SemaphoreType.DMA((NSCHED, 4)),
                pltpu.SemaphoreType.DMA((NSCHED, 4)),
                pltpu.SemaphoreType.DMA((NSCHED, 4)),
            ]
        ),
        compiler_params=pltpu.CompilerParams(collective_id=0),
    )(x, wq_loc, K_ext, V_ext, wo_loc)


# device time: 58003 ns/iter; 1.3396x vs baseline; 1.0542x over previous
import jax
import jax.numpy as jnp
from jax import lax
from jax.experimental import pallas as pl
from jax.experimental.pallas import tpu as pltpu

N = 16
B = 2
SQ = 512
HL = 8
DH = 64
D_MODEL = 768
D_LOC = HL * DH
ROWS = B * SQ
NSCHED = 2
CW = D_MODEL // NSCHED

SCHED_STAGES = [
    [(1, 0), (4, 2), (2, 1), (8, 3)],
    [(4, 2), (1, 0), (8, 3), (2, 1)],
]
HALVES = [ROWS >> (k + 1) for k in range(4)]


def _body(x_ref, wq_ref, k_ref, v_ref, wo_ref, out_ref,
          acc_ref, agb_ref,
          sb0, sb1, sb2, sb3, sb4, sb5, sb6, sb7,
          rx0, rx1, rx2, rx3, rx4, rx5, rx6, rx7,
          rs_ssem, rs_rsem, ag_ssem, ag_rsem):
    me = lax.axis_index("i")
    sb = [[sb0, sb1, sb2, sb3], [sb4, sb5, sb6, sb7]]
    rx = [[rx0, rx1, rx2, rx3], [rx4, rx5, rx6, rx7]]

    qb = lax.broadcasted_iota(jnp.int32, (SQ, SQ), 0) // 64
    kb = lax.broadcasted_iota(jnp.int32, (SQ, SQ), 1) // 64
    bias = jnp.where(kb <= qb, 0.0, -1e9).astype(jnp.float32)
    for b in range(B):
        q = jnp.dot(x_ref[b], wq_ref[:, :],
                    preferred_element_type=jnp.float32)
        ctx_parts = []
        for h in range(HL):
            qh = q[:, h * DH:(h + 1) * DH]
            kh = k_ref[b, :, h, :]
            s = jnp.dot(qh, kh.T,
                        preferred_element_type=jnp.float32) * 0.125 + bias
            w = jnp.exp(s)
            denom = jnp.sum(w, axis=-1, keepdims=True)
            ctx_parts.append(
                jnp.dot(w, v_ref[b, :, h, :],
                        preferred_element_type=jnp.float32) / denom)
        ctx = jnp.concatenate(ctx_parts, axis=1)
        acc_ref[b * SQ:(b + 1) * SQ, :] = jnp.dot(
            ctx, wo_ref[:, :], preferred_element_type=jnp.float32)

    barrier = pltpu.get_barrier_semaphore()
    for m in (1, 2, 4, 8):
        pl.semaphore_signal(barrier, inc=1, device_id=(me ^ m,),
                            device_id_type=pl.DeviceIdType.MESH)
    pl.semaphore_wait(barrier, 4)

    base = [jnp.int32(0), jnp.int32(0)]
    for k in range(4):
        half = HALVES[k]
        rdmas = []
        for s in range(NSCHED):
            m, shift = SCHED_STAGES[s][k]
            bit = (me >> shift) & 1
            send_off = pl.multiple_of(base[s] + (1 - bit) * half, 64)
            sb[s][k][...] = acc_ref[
                pl.ds(send_off, half), s * CW:(s + 1) * CW].astype(jnp.bfloat16)
            rdma = pltpu.make_async_remote_copy(
                src_ref=sb[s][k],
                dst_ref=rx[s][k],
                send_sem=rs_ssem.at[s, k],
                recv_sem=rs_rsem.at[s, k],
                device_id=(me ^ m,),
                device_id_type=pl.DeviceIdType.MESH,
            )
            rdma.start()
            base[s] = pl.multiple_of(base[s] + bit * half, 64)
            rdmas.append(rdma)
        for s in range(NSCHED):
            rdmas[s].wait()
            acc_ref[pl.ds(base[s], half), s * CW:(s + 1) * CW] = (
                acc_ref[pl.ds(base[s], half), s * CW:(s + 1) * CW]
                + rx[s][k][...].astype(jnp.float32))

    for s in range(NSCHED):
        agb_ref[pl.ds(base[s], HALVES[3]), s * CW:(s + 1) * CW] = acc_ref[
            pl.ds(base[s], HALVES[3]), s * CW:(s + 1) * CW].astype(jnp.bfloat16)
    for k in range(4):
        seg = HALVES[3 - k]
        rdmas = []
        for s in range(NSCHED):
            m, _ = SCHED_STAGES[s][3 - k]
            rdma = pltpu.make_async_remote_copy(
                src_ref=agb_ref.at[pl.ds(base[s], seg), s * CW:(s + 1) * CW],
                dst_ref=agb_ref.at[pl.ds(base[s], seg), s * CW:(s + 1) * CW],
                send_sem=ag_ssem.at[s, k],
                recv_sem=ag_rsem.at[s, k],
                device_id=(me ^ m,),
                device_id_type=pl.DeviceIdType.MESH,
            )
            rdma.start()
            rdmas.append(rdma)
        for s in range(NSCHED):
            rdmas[s].wait()
            base[s] = pl.multiple_of(base[s] - (base[s] & seg), 64)

    out_ref[0] = agb_ref[0:SQ, :].astype(jnp.float32)
    out_ref[1] = agb_ref[SQ:ROWS, :].astype(jnp.float32)


def kernel(x, Wq, K_ext, V_ext, Wo):
    me = lax.axis_index("i")
    wq_loc = lax.dynamic_slice(Wq, (0, me * D_LOC), (Wq.shape[0], D_LOC))
    wo_loc = lax.dynamic_slice(Wo, (me * D_LOC, 0), (D_LOC, Wo.shape[1]))

    stage_bufs = [
        pltpu.VMEM((HALVES[k], CW), jnp.bfloat16)
        for _ in range(NSCHED) for k in range(4)
    ]
    return pl.pallas_call(
        _body,
        out_shape=jax.ShapeDtypeStruct((B, SQ, D_MODEL), jnp.float32),
        in_specs=[pl.BlockSpec(memory_space=pltpu.VMEM)] * 5,
        out_specs=pl.BlockSpec(memory_space=pltpu.VMEM),
        scratch_shapes=(
            [
                pltpu.VMEM((ROWS, D_MODEL), jnp.float32),
                pltpu.VMEM((ROWS, D_MODEL), jnp.bfloat16),
            ]
            + stage_bufs
            + stage_bufs
            + [
                pltpu.SemaphoreType.DMA((NSCHED, 4)),
                pltpu.SemaphoreType.DMA((NSCHED, 4)),
                pltpu.SemaphoreType.DMA((NSCHED, 4)),
                pltpu.SemaphoreType.DMA((NSCHED, 4)),
            ]
        ),
        compiler_params=pltpu.CompilerParams(collective_id=0),
    )(x, wq_loc, K_ext, V_ext, wo_loc)


# device time: 56850 ns/iter; 1.3667x vs baseline; 1.0203x over previous
import jax
import jax.numpy as jnp
from jax import lax
from jax.experimental import pallas as pl
from jax.experimental.pallas import tpu as pltpu

N = 16
B = 2
SQ = 512
HL = 8
DH = 64
D_MODEL = 768
D_LOC = HL * DH
ROWS = B * SQ
NSCHED = 2
CW = D_MODEL // NSCHED

SCHED_STAGES = [
    [(1, 0), (4, 2), (2, 1), (8, 3)],
    [(4, 2), (1, 0), (8, 3), (2, 1)],
]
HALVES = [ROWS >> (k + 1) for k in range(4)]


def _body(x_ref, wq_ref, k_ref, v_ref, wo_ref, out_ref,
          acc_ref, agb_ref,
          sb0, sb1, sb2, sb3, sb4, sb5, sb6, sb7,
          rx0, rx1, rx2, rx3, rx4, rx5, rx6, rx7,
          rs_ssem, rs_rsem, ag_ssem, ag_rsem):
    me = lax.axis_index("i")
    sb = [[sb0, sb1, sb2, sb3], [sb4, sb5, sb6, sb7]]
    rx = [[rx0, rx1, rx2, rx3], [rx4, rx5, rx6, rx7]]

    barrier = pltpu.get_barrier_semaphore()
    for m in (1, 2, 4, 8):
        pl.semaphore_signal(barrier, inc=1, device_id=(me ^ m,),
                            device_id_type=pl.DeviceIdType.MESH)
    pl.semaphore_wait(barrier, 4)

    HQ = SQ // 2
    qb = lax.broadcasted_iota(jnp.int32, (SQ, SQ), 0) // 64
    kb = lax.broadcasted_iota(jnp.int32, (SQ, SQ), 1) // 64
    full_bias = jnp.where(kb <= qb, 0.0, -1e9).astype(jnp.float32)
    bias_lo = full_bias[0:HQ, 0:HQ]
    bias_hi = full_bias[HQ:SQ, :]
    for b in range(B):
        q = jnp.dot(x_ref[b], wq_ref[:, :],
                    preferred_element_type=jnp.float32)
        ctx_parts = []
        for h in range(HL):
            qh = q[:, h * DH:(h + 1) * DH]
            kh = k_ref[b, :, h, :]
            vh = v_ref[b, :, h, :]
            s_lo = jnp.dot(qh[0:HQ, :], kh[0:HQ, :].T,
                           preferred_element_type=jnp.float32) * 0.125
            w_lo = jnp.exp(s_lo + bias_lo)
            ctx_lo = (jnp.dot(w_lo, vh[0:HQ, :],
                              preferred_element_type=jnp.float32)
                      / jnp.sum(w_lo, axis=-1, keepdims=True))
            s_hi = jnp.dot(qh[HQ:SQ, :], kh.T,
                           preferred_element_type=jnp.float32) * 0.125
            w_hi = jnp.exp(s_hi + bias_hi)
            ctx_hi = (jnp.dot(w_hi, vh,
                              preferred_element_type=jnp.float32)
                      / jnp.sum(w_hi, axis=-1, keepdims=True))
            ctx_parts.append(jnp.concatenate([ctx_lo, ctx_hi], axis=0))
        ctx = jnp.concatenate(ctx_parts, axis=1)
        acc_ref[b * SQ:(b + 1) * SQ, :] = jnp.dot(
            ctx, wo_ref[:, :], preferred_element_type=jnp.float32)

    base = [jnp.int32(0), jnp.int32(0)]
    for k in range(4):
        half = HALVES[k]
        rdmas = []
        for s in range(NSCHED):
            m, shift = SCHED_STAGES[s][k]
            bit = (me >> shift) & 1
            send_off = pl.multiple_of(base[s] + (1 - bit) * half, 64)
            sb[s][k][...] = acc_ref[
                pl.ds(send_off, half), s * CW:(s + 1) * CW].astype(jnp.bfloat16)
            rdma = pltpu.make_async_remote_copy(
                src_ref=sb[s][k],
                dst_ref=rx[s][k],
                send_sem=rs_ssem.at[s, k],
                recv_sem=rs_rsem.at[s, k],
                device_id=(me ^ m,),
                device_id_type=pl.DeviceIdType.MESH,
            )
            rdma.start()
            base[s] = pl.multiple_of(base[s] + bit * half, 64)
            rdmas.append(rdma)
        for s in range(NSCHED):
            rdmas[s].wait()
            acc_ref[pl.ds(base[s], half), s * CW:(s + 1) * CW] = (
                acc_ref[pl.ds(base[s], half), s * CW:(s + 1) * CW]
                + rx[s][k][...].astype(jnp.float32))

    for s in range(NSCHED):
        agb_ref[pl.ds(base[s], HALVES[3]), s * CW:(s + 1) * CW] = acc_ref[
            pl.ds(base[s], HALVES[3]), s * CW:(s + 1) * CW].astype(jnp.bfloat16)
    for k in range(4):
        seg = HALVES[3 - k]
        rdmas = []
        for s in range(NSCHED):
            m, _ = SCHED_STAGES[s][3 - k]
            rdma = pltpu.make_async_remote_copy(
                src_ref=agb_ref.at[pl.ds(base[s], seg), s * CW:(s + 1) * CW],
                dst_ref=agb_ref.at[pl.ds(base[s], seg), s * CW:(s + 1) * CW],
                send_sem=ag_ssem.at[s, k],
                recv_sem=ag_rsem.at[s, k],
                device_id=(me ^ m,),
                device_id_type=pl.DeviceIdType.MESH,
            )
            rdma.start()
            rdmas.append(rdma)
        for s in range(NSCHED):
            rdmas[s].wait()
            base[s] = pl.multiple_of(base[s] - (base[s] & seg), 64)

    out_ref[0] = agb_ref[0:SQ, :].astype(jnp.float32)
    out_ref[1] = agb_ref[SQ:ROWS, :].astype(jnp.float32)


def kernel(x, Wq, K_ext, V_ext, Wo):
    me = lax.axis_index("i")
    wq_loc = lax.dynamic_slice(Wq, (0, me * D_LOC), (Wq.shape[0], D_LOC))
    wo_loc = lax.dynamic_slice(Wo, (me * D_LOC, 0), (D_LOC, Wo.shape[1]))

    stage_bufs = [
        pltpu.VMEM((HALVES[k], CW), jnp.bfloat16)
        for _ in range(NSCHED) for k in range(4)
    ]
    return pl.pallas_call(
        _body,
        out_shape=jax.ShapeDtypeStruct((B, SQ, D_MODEL), jnp.float32),
        in_specs=[pl.BlockSpec(memory_space=pltpu.VMEM)] * 5,
        out_specs=pl.BlockSpec(memory_space=pltpu.VMEM),
        scratch_shapes=(
            [
                pltpu.VMEM((ROWS, D_MODEL), jnp.float32),
                pltpu.VMEM((ROWS, D_MODEL), jnp.bfloat16),
            ]
            + stage_bufs
            + stage_bufs
            + [
                pltpu.SemaphoreType.DMA((NSCHED, 4)),
                pltpu.SemaphoreType.DMA((NSCHED, 4)),
                pltpu.SemaphoreType.DMA((NSCHED, 4)),
                pltpu.SemaphoreType.DMA((NSCHED, 4)),
            ]
        ),
        compiler_params=pltpu.CompilerParams(collective_id=0),
    )(x, wq_loc, K_ext, V_ext, wo_loc)


# device time: 50405 ns/iter; 1.5415x vs baseline; 1.1279x over previous
import jax
import jax.numpy as jnp
from jax import lax
from jax.experimental import pallas as pl
from jax.experimental.pallas import tpu as pltpu

N = 16
B = 2
SQ = 512
HL = 8
DH = 64
D_MODEL = 768
D_LOC = HL * DH
ROWS = B * SQ
NSCHED = 2
CW = D_MODEL // NSCHED

SCHED_STAGES = [
    [(1, 0), (4, 2), (2, 1), (8, 3)],
    [(4, 2), (1, 0), (8, 3), (2, 1)],
]
HALVES = [SQ >> (k + 1) for k in range(4)]


class _BatchAllReduce:

    def __init__(self, b, me, acc_ref, agb_ref, sb, rx, sems):
        self.b = b
        self.me = me
        self.acc = acc_ref
        self.agb = agb_ref
        self.sb = sb[b]
        self.rx = rx[b]
        self.rs_ssem, self.rs_rsem, self.ag_ssem, self.ag_rsem = sems
        self.base = [jnp.int32(b * SQ), jnp.int32(b * SQ)]
        self.rdmas = [None, None]

    def _cols(self, s):
        return slice(s * CW, (s + 1) * CW)

    def start_rs(self, k):
        half = HALVES[k]
        for s in range(NSCHED):
            m, shift = SCHED_STAGES[s][k]
            bit = (self.me >> shift) & 1
            send_off = pl.multiple_of(self.base[s] + (1 - bit) * half, 32)
            self.sb[s][k][...] = self.acc[
                pl.ds(send_off, half), self._cols(s)].astype(jnp.bfloat16)
            rdma = pltpu.make_async_remote_copy(
                src_ref=self.sb[s][k],
                dst_ref=self.rx[s][k],
                send_sem=self.rs_ssem.at[self.b, s, k],
                recv_sem=self.rs_rsem.at[self.b, s, k],
                device_id=(self.me ^ m,),
                device_id_type=pl.DeviceIdType.MESH,
            )
            rdma.start()
            self.base[s] = pl.multiple_of(self.base[s] + bit * half, 32)
            self.rdmas[s] = rdma

    def finish_rs(self, k):
        half = HALVES[k]
        for s in range(NSCHED):
            self.rdmas[s].wait()
            self.acc[pl.ds(self.base[s], half), self._cols(s)] = (
                self.acc[pl.ds(self.base[s], half), self._cols(s)]
                + self.rx[s][k][...].astype(jnp.float32))

    def seed_ag(self):
        for s in range(NSCHED):
            self.agb[pl.ds(self.base[s], HALVES[3]), self._cols(s)] = (
                self.acc[pl.ds(self.base[s], HALVES[3]),
                         self._cols(s)].astype(jnp.bfloat16))

    def start_ag(self, k):
        seg = HALVES[3 - k]
        for s in range(NSCHED):
            m, _ = SCHED_STAGES[s][3 - k]
            rdma = pltpu.make_async_remote_copy(
                src_ref=self.agb.at[pl.ds(self.base[s], seg), self._cols(s)],
                dst_ref=self.agb.at[pl.ds(self.base[s], seg), self._cols(s)],
                send_sem=self.ag_ssem.at[self.b, s, k],
                recv_sem=self.ag_rsem.at[self.b, s, k],
                device_id=(self.me ^ m,),
                device_id_type=pl.DeviceIdType.MESH,
            )
            rdma.start()
            self.rdmas[s] = rdma

    def finish_ag(self, k):
        seg = HALVES[3 - k]
        for s in range(NSCHED):
            self.rdmas[s].wait()
            local = self.base[s] - self.b * SQ
            self.base[s] = pl.multiple_of(
                self.b * SQ + local - (local & seg), 32)


def _compute_batch(b, x_ref, wq_ref, k_ref, v_ref, wo_ref, acc_ref, bias):
    HQ = SQ // 2
    bias_lo = bias[0:HQ, 0:HQ]
    bias_hi = bias[HQ:SQ, :]
    q = jnp.dot(x_ref[b], wq_ref[:, :],
                preferred_element_type=jnp.float32)
    ctx_parts = []
    for h in range(HL):
        qh = q[:, h * DH:(h + 1) * DH]
        kh = k_ref[b, :, h, :]
        vh = v_ref[b, :, h, :]
        s_lo = jnp.dot(qh[0:HQ, :], kh[0:HQ, :].T,
                       preferred_element_type=jnp.float32) * 0.125
        w_lo = jnp.exp(s_lo + bias_lo)
        ctx_lo = (jnp.dot(w_lo, vh[0:HQ, :],
                          preferred_element_type=jnp.float32)
                  / jnp.sum(w_lo, axis=-1, keepdims=True))
        s_hi = jnp.dot(qh[HQ:SQ, :], kh.T,
                       preferred_element_type=jnp.float32) * 0.125
        w_hi = jnp.exp(s_hi + bias_hi)
        ctx_hi = (jnp.dot(w_hi, vh,
                          preferred_element_type=jnp.float32)
                  / jnp.sum(w_hi, axis=-1, keepdims=True))
        ctx_parts.append(jnp.concatenate([ctx_lo, ctx_hi], axis=0))
    ctx = jnp.concatenate(ctx_parts, axis=1)
    acc_ref[b * SQ:(b + 1) * SQ, :] = jnp.dot(
        ctx, wo_ref[:, :], preferred_element_type=jnp.float32)


def _body(x_ref, wq_ref, k_ref, v_ref, wo_ref, out_ref,
          acc_ref, agb_ref,
          sb000, sb001, sb002, sb003, sb010, sb011, sb012, sb013,
          sb100, sb101, sb102, sb103, sb110, sb111, sb112, sb113,
          rx000, rx001, rx002, rx003, rx010, rx011, rx012, rx013,
          rx100, rx101, rx102, rx103, rx110, rx111, rx112, rx113,
          rs_ssem, rs_rsem, ag_ssem, ag_rsem):
    me = lax.axis_index("i")
    sb = [[[sb000, sb001, sb002, sb003], [sb010, sb011, sb012, sb013]],
          [[sb100, sb101, sb102, sb103], [sb110, sb111, sb112, sb113]]]
    rx = [[[rx000, rx001, rx002, rx003], [rx010, rx011, rx012, rx013]],
          [[rx100, rx101, rx102, rx103], [rx110, rx111, rx112, rx113]]]

    barrier = pltpu.get_barrier_semaphore()
    for m in (1, 2, 4, 8):
        pl.semaphore_signal(barrier, inc=1, device_id=(me ^ m,),
                            device_id_type=pl.DeviceIdType.MESH)
    pl.semaphore_wait(barrier, 4)

    qb = lax.broadcasted_iota(jnp.int32, (SQ, SQ), 0) // 64
    kb = lax.broadcasted_iota(jnp.int32, (SQ, SQ), 1) // 64
    bias = jnp.where(kb <= qb, 0.0, -1e9).astype(jnp.float32)

    sems = (rs_ssem, rs_rsem, ag_ssem, ag_rsem)
    ar0 = _BatchAllReduce(0, me, acc_ref, agb_ref, sb, rx, sems)
    ar1 = _BatchAllReduce(1, me, acc_ref, agb_ref, sb, rx, sems)

    _compute_batch(0, x_ref, wq_ref, k_ref, v_ref, wo_ref, acc_ref, bias)
    ar0.start_rs(0)
    _compute_batch(1, x_ref, wq_ref, k_ref, v_ref, wo_ref, acc_ref, bias)

    ar0.finish_rs(0); ar0.start_rs(1)
    ar1.start_rs(0)
    ar0.finish_rs(1); ar0.start_rs(2)
    ar1.finish_rs(0); ar1.start_rs(1)
    ar0.finish_rs(2); ar0.start_rs(3)
    ar1.finish_rs(1); ar1.start_rs(2)
    ar0.finish_rs(3); ar0.seed_ag(); ar0.start_ag(0)
    ar1.finish_rs(2); ar1.start_rs(3)
    ar0.finish_ag(0); ar0.start_ag(1)
    ar1.finish_rs(3); ar1.seed_ag(); ar1.start_ag(0)
    ar0.finish_ag(1); ar0.start_ag(2)
    ar1.finish_ag(0); ar1.start_ag(1)
    ar0.finish_ag(2); ar0.start_ag(3)
    ar1.finish_ag(1); ar1.start_ag(2)
    ar0.finish_ag(3)
    out_ref[0] = agb_ref[0:SQ, :].astype(jnp.float32)
    ar1.finish_ag(2); ar1.start_ag(3)
    ar1.finish_ag(3)
    out_ref[1] = agb_ref[SQ:ROWS, :].astype(jnp.float32)


def kernel(x, Wq, K_ext, V_ext, Wo):
    me = lax.axis_index("i")
    wq_loc = lax.dynamic_slice(Wq, (0, me * D_LOC), (Wq.shape[0], D_LOC))
    wo_loc = lax.dynamic_slice(Wo, (me * D_LOC, 0), (D_LOC, Wo.shape[1]))

    stage_bufs = [
        pltpu.VMEM((HALVES[k], CW), jnp.bfloat16)
        for _ in range(B) for _ in range(NSCHED) for k in range(4)
    ]
    return pl.pallas_call(
        _body,
        out_shape=jax.ShapeDtypeStruct((B, SQ, D_MODEL), jnp.float32),
        in_specs=[pl.BlockSpec(memory_space=pltpu.VMEM)] * 5,
        out_specs=pl.BlockSpec(memory_space=pltpu.VMEM),
        scratch_shapes=(
            [
                pltpu.VMEM((ROWS, D_MODEL), jnp.float32),
                pltpu.VMEM((ROWS, D_MODEL), jnp.bfloat16),
            ]
            + stage_bufs
            + stage_bufs
            + [
                pltpu.SemaphoreType.DMA((B, NSCHED, 4)),
                pltpu.SemaphoreType.DMA((B, NSCHED, 4)),
                pltpu.SemaphoreType.DMA((B, NSCHED, 4)),
                pltpu.SemaphoreType.DMA((B, NSCHED, 4)),
            ]
        ),
        compiler_params=pltpu.CompilerParams(collective_id=0),
    )(x, wq_loc, K_ext, V_ext, wo_loc)


# device time: 45402 ns/iter; 1.7113x vs baseline; 1.1102x over previous
import jax
import jax.numpy as jnp
from jax import lax
from jax.experimental import pallas as pl
from jax.experimental.pallas import tpu as pltpu

N = 16
B = 2
SQ = 512
HL = 8
DH = 64
D_MODEL = 768
D_LOC = HL * DH
ROWS = B * SQ
NSCHED = 2
CW = D_MODEL // NSCHED
QR = SQ // 4
SR = QR // 4

SCHED_KINDS = [(0, 1), (1, 0)]


def _member(me, kind, idx):
    if kind == 0:
        return (me & ~3) + idx
    return (me & 3) + 4 * idx


class _BatchAllReduce:

    def __init__(self, b, me, acc_ref, agb_ref, sbq, rxq, sbs, rxs, sems):
        self.b = b
        self.me = me
        self.acc = acc_ref
        self.agb = agb_ref
        self.sbq, self.rxq = sbq, rxq
        self.sbs, self.rxs = sbs, rxs
        (self.rs0_s, self.rs0_r, self.rs1_s, self.rs1_r,
         self.ag0_s, self.ag0_r, self.ag1_s, self.ag1_r) = sems
        i = me & 3
        z = me >> 2
        self.g0 = [i if SCHED_KINDS[s][0] == 0 else z for s in range(NSCHED)]
        self.g1 = [z if SCHED_KINDS[s][0] == 0 else i for s in range(NSCHED)]
        self.q_off = [pl.multiple_of(b * SQ + self.g0[s] * QR, QR)
                      for s in range(NSCHED)]
        self.f_off = [pl.multiple_of(self.q_off[s] + self.g1[s] * SR, SR)
                      for s in range(NSCHED)]
        self.rdmas = [[None] * 3, [None] * 3]

    def _cols(self, s):
        return slice(s * CW, (s + 1) * CW)

    def start_rs0(self):
        for s in range(NSCHED):
            kind = SCHED_KINDS[s][0]
            for d in range(1, 4):
                tgt = (self.g0[s] + d) % 4
                off = pl.multiple_of(self.b * SQ + tgt * QR, QR)
                self.sbq[self.b, s, d - 1, :, :] = self.acc[
                    pl.ds(off, QR), self._cols(s)].astype(jnp.bfloat16)
                rdma = pltpu.make_async_remote_copy(
                    src_ref=self.sbq.at[self.b, s, d - 1],
                    dst_ref=self.rxq.at[self.b, s, d - 1],
                    send_sem=self.rs0_s.at[self.b, s, d - 1],
                    recv_sem=self.rs0_r.at[self.b, s, d - 1],
                    device_id=(_member(self.me, kind, tgt),),
                    device_id_type=pl.DeviceIdType.MESH,
                )
                rdma.start()
                self.rdmas[s][d - 1] = rdma

    def finish_rs0(self):
        for s in range(NSCHED):
            for d in range(3):
                self.rdmas[s][d].wait()
            self.acc[pl.ds(self.q_off[s], QR), self._cols(s)] = (
                self.acc[pl.ds(self.q_off[s], QR), self._cols(s)]
                + self.rxq[self.b, s, 0].astype(jnp.float32)
                + self.rxq[self.b, s, 1].astype(jnp.float32)
                + self.rxq[self.b, s, 2].astype(jnp.float32))

    def start_rs1(self):
        for s in range(NSCHED):
            kind = SCHED_KINDS[s][1]
            for d in range(1, 4):
                tgt = (self.g1[s] + d) % 4
                off = pl.multiple_of(self.q_off[s] + tgt * SR, SR)
                self.sbs[self.b, s, d - 1, :, :] = self.acc[
                    pl.ds(off, SR), self._cols(s)].astype(jnp.bfloat16)
                rdma = pltpu.make_async_remote_copy(
                    src_ref=self.sbs.at[self.b, s, d - 1],
                    dst_ref=self.rxs.at[self.b, s, d - 1],
                    send_sem=self.rs1_s.at[self.b, s, d - 1],
                    recv_sem=self.rs1_r.at[self.b, s, d - 1],
                    device_id=(_member(self.me, kind, tgt),),
                    device_id_type=pl.DeviceIdType.MESH,
                )
                rdma.start()
                self.rdmas[s][d - 1] = rdma

    def finish_rs1(self):
        for s in range(NSCHED):
            for d in range(3):
                self.rdmas[s][d].wait()
            self.acc[pl.ds(self.f_off[s], SR), self._cols(s)] = (
                self.acc[pl.ds(self.f_off[s], SR), self._cols(s)]
                + self.rxs[self.b, s, 0].astype(jnp.float32)
                + self.rxs[self.b, s, 1].astype(jnp.float32)
                + self.rxs[self.b, s, 2].astype(jnp.float32))

    def start_ag0(self):
        for s in range(NSCHED):
            kind = SCHED_KINDS[s][1]
            self.agb[pl.ds(self.f_off[s], SR), self._cols(s)] = self.acc[
                pl.ds(self.f_off[s], SR), self._cols(s)].astype(jnp.bfloat16)
            for d in range(1, 4):
                tgt = (self.g1[s] + d) % 4
                rdma = pltpu.make_async_remote_copy(
                    src_ref=self.agb.at[pl.ds(self.f_off[s], SR),
                                        self._cols(s)],
                    dst_ref=self.agb.at[pl.ds(self.f_off[s], SR),
                                        self._cols(s)],
                    send_sem=self.ag0_s.at[self.b, s, d - 1],
                    recv_sem=self.ag0_r.at[self.b, s, d - 1],
                    device_id=(_member(self.me, kind, tgt),),
                    device_id_type=pl.DeviceIdType.MESH,
                )
                rdma.start()
                self.rdmas[s][d - 1] = rdma

    def finish_ag0(self):
        for s in range(NSCHED):
            for d in range(3):
                self.rdmas[s][d].wait()

    def start_ag1(self):
        for s in range(NSCHED):
            kind = SCHED_KINDS[s][0]
            for d in range(1, 4):
                tgt = (self.g0[s] + d) % 4
                rdma = pltpu.make_async_remote_copy(
                    src_ref=self.agb.at[pl.ds(self.q_off[s], QR),
                                        self._cols(s)],
                    dst_ref=self.agb.at[pl.ds(self.q_off[s], QR),
                                        self._cols(s)],
                    send_sem=self.ag1_s.at[self.b, s, d - 1],
                    recv_sem=self.ag1_r.at[self.b, s, d - 1],
                    device_id=(_member(self.me, kind, tgt),),
                    device_id_type=pl.DeviceIdType.MESH,
                )
                rdma.start()
                self.rdmas[s][d - 1] = rdma

    def finish_ag1(self):
        for s in range(NSCHED):
            for d in range(3):
                self.rdmas[s][d].wait()


def _compute_batch(b, x_ref, wq_ref, k_ref, v_ref, wo_ref, acc_ref, bias):
    HQ = SQ // 2
    bias_lo = bias[0:HQ, 0:HQ]
    bias_hi = bias[HQ:SQ, :]
    q = jnp.dot(x_ref[b], wq_ref[:, :],
                preferred_element_type=jnp.float32)
    ctx_parts = []
    for h in range(HL):
        qh = q[:, h * DH:(h + 1) * DH]
        kh = k_ref[b, :, h, :]
        vh = v_ref[b, :, h, :]
        s_lo = jnp.dot(qh[0:HQ, :], kh[0:HQ, :].T,
                       preferred_element_type=jnp.float32) * 0.125
        w_lo = jnp.exp(s_lo + bias_lo)
        ctx_lo = (jnp.dot(w_lo, vh[0:HQ, :],
                          preferred_element_type=jnp.float32)
                  / jnp.sum(w_lo, axis=-1, keepdims=True))
        s_hi = jnp.dot(qh[HQ:SQ, :], kh.T,
                       preferred_element_type=jnp.float32) * 0.125
        w_hi = jnp.exp(s_hi + bias_hi)
        ctx_hi = (jnp.dot(w_hi, vh,
                          preferred_element_type=jnp.float32)
                  / jnp.sum(w_hi, axis=-1, keepdims=True))
        ctx_parts.append(jnp.concatenate([ctx_lo, ctx_hi], axis=0))
    ctx = jnp.concatenate(ctx_parts, axis=1)
    acc_ref[b * SQ:(b + 1) * SQ, :] = jnp.dot(
        ctx, wo_ref[:, :], preferred_element_type=jnp.float32)


def _body(x_ref, wq_ref, k_ref, v_ref, wo_ref, out_ref,
          acc_ref, agb_ref, sbq, rxq, sbs, rxs,
          rs0_s, rs0_r, rs1_s, rs1_r, ag0_s, ag0_r, ag1_s, ag1_r):
    me = lax.axis_index("i")

    barrier = pltpu.get_barrier_semaphore()
    i = me & 3
    z = me >> 2
    for d in range(1, 4):
        pl.semaphore_signal(barrier, inc=1,
                            device_id=(_member(me, 0, (i + d) % 4),),
                            device_id_type=pl.DeviceIdType.MESH)
        pl.semaphore_signal(barrier, inc=1,
                            device_id=(_member(me, 1, (z + d) % 4),),
                            device_id_type=pl.DeviceIdType.MESH)
    pl.semaphore_wait(barrier, 6)

    qb = lax.broadcasted_iota(jnp.int32, (SQ, SQ), 0) // 64
    kb = lax.broadcasted_iota(jnp.int32, (SQ, SQ), 1) // 64
    bias = jnp.where(kb <= qb, 0.0, -1e9).astype(jnp.float32)

    sems = (rs0_s, rs0_r, rs1_s, rs1_r, ag0_s, ag0_r, ag1_s, ag1_r)
    ar0 = _BatchAllReduce(0, me, acc_ref, agb_ref, sbq, rxq, sbs, rxs, sems)
    ar1 = _BatchAllReduce(1, me, acc_ref, agb_ref, sbq, rxq, sbs, rxs, sems)

    _compute_batch(0, x_ref, wq_ref, k_ref, v_ref, wo_ref, acc_ref, bias)
    ar0.start_rs0()
    _compute_batch(1, x_ref, wq_ref, k_ref, v_ref, wo_ref, acc_ref, bias)

    ar0.finish_rs0(); ar0.start_rs1()
    ar1.start_rs0()
    ar0.finish_rs1(); ar0.start_ag0()
    ar1.finish_rs0(); ar1.start_rs1()
    ar0.finish_ag0(); ar0.start_ag1()
    ar1.finish_rs1(); ar1.start_ag0()
    ar0.finish_ag1()
    out_ref[0] = agb_ref[0:SQ, :].astype(jnp.float32)
    ar1.finish_ag0(); ar1.start_ag1()
    ar1.finish_ag1()
    out_ref[1] = agb_ref[SQ:ROWS, :].astype(jnp.float32)


def kernel(x, Wq, K_ext, V_ext, Wo):
    me = lax.axis_index("i")
    wq_loc = lax.dynamic_slice(Wq, (0, me * D_LOC), (Wq.shape[0], D_LOC))
    wo_loc = lax.dynamic_slice(Wo, (me * D_LOC, 0), (D_LOC, Wo.shape[1]))

    return pl.pallas_call(
        _body,
        out_shape=jax.ShapeDtypeStruct((B, SQ, D_MODEL), jnp.float32),
        in_specs=[pl.BlockSpec(memory_space=pltpu.VMEM)] * 5,
        out_specs=pl.BlockSpec(memory_space=pltpu.VMEM),
        scratch_shapes=[
            pltpu.VMEM((ROWS, D_MODEL), jnp.float32),
            pltpu.VMEM((ROWS, D_MODEL), jnp.bfloat16),
            pltpu.VMEM((B, NSCHED, 3, QR, CW), jnp.bfloat16),
            pltpu.VMEM((B, NSCHED, 3, QR, CW), jnp.bfloat16),
            pltpu.VMEM((B, NSCHED, 3, SR, CW), jnp.bfloat16),
            pltpu.VMEM((B, NSCHED, 3, SR, CW), jnp.bfloat16),
            pltpu.SemaphoreType.DMA((B, NSCHED, 3)),
            pltpu.SemaphoreType.DMA((B, NSCHED, 3)),
            pltpu.SemaphoreType.DMA((B, NSCHED, 3)),
            pltpu.SemaphoreType.DMA((B, NSCHED, 3)),
            pltpu.SemaphoreType.DMA((B, NSCHED, 3)),
            pltpu.SemaphoreType.DMA((B, NSCHED, 3)),
            pltpu.SemaphoreType.DMA((B, NSCHED, 3)),
            pltpu.SemaphoreType.DMA((B, NSCHED, 3)),
        ],
        compiler_params=pltpu.CompilerParams(collective_id=0),
    )(x, wq_loc, K_ext, V_ext, wo_loc)


# device time: 44172 ns/iter; 1.7590x vs baseline; 1.0278x over previous
import jax
import jax.numpy as jnp
from jax import lax
from jax.experimental import pallas as pl
from jax.experimental.pallas import tpu as pltpu

N = 16
B = 2
SQ = 512
HL = 8
DH = 64
D_MODEL = 768
D_LOC = HL * DH
ROWS = B * SQ
NSCHED = 2
QR = SQ // 4
SR = QR // 4

SCHED_KINDS = [(0, 1), (1, 0)]
COLS = ((0, 512), (512, 256))


def _member(me, kind, idx):
    if kind == 0:
        return (me & ~3) + idx
    return (me & 3) + 4 * idx


class _BatchAllReduce:

    def __init__(self, b, me, acc_ref, agb_ref, sbq, rxq, sbs, rxs, sems):
        self.b = b
        self.me = me
        self.acc = acc_ref
        self.agb = agb_ref
        self.sbq, self.rxq = sbq, rxq
        self.sbs, self.rxs = sbs, rxs
        (self.rs0_s, self.rs0_r, self.rs1_s, self.rs1_r,
         self.ag0_s, self.ag0_r, self.ag1_s, self.ag1_r) = sems
        i = me & 3
        z = me >> 2
        self.g0 = [i if SCHED_KINDS[s][0] == 0 else z for s in range(NSCHED)]
        self.g1 = [z if SCHED_KINDS[s][0] == 0 else i for s in range(NSCHED)]
        self.q_off = [pl.multiple_of(b * SQ + self.g0[s] * QR, QR)
                      for s in range(NSCHED)]
        self.f_off = [pl.multiple_of(self.q_off[s] + self.g1[s] * SR, SR)
                      for s in range(NSCHED)]
        self.rdmas = [[None] * 3, [None] * 3]

    def _cols(self, s):
        return slice(COLS[s][0], COLS[s][0] + COLS[s][1])

    def start_rs0(self):
        for s in range(NSCHED):
            kind = SCHED_KINDS[s][0]
            for d in range(1, 4):
                tgt = (self.g0[s] + d) % 4
                off = pl.multiple_of(self.b * SQ + tgt * QR, QR)
                self.sbq[s][self.b, d - 1, :, :] = self.acc[
                    pl.ds(off, QR), self._cols(s)].astype(jnp.bfloat16)
                rdma = pltpu.make_async_remote_copy(
                    src_ref=self.sbq[s].at[self.b, d - 1],
                    dst_ref=self.rxq[s].at[self.b, d - 1],
                    send_sem=self.rs0_s.at[self.b, s, d - 1],
                    recv_sem=self.rs0_r.at[self.b, s, d - 1],
                    device_id=(_member(self.me, kind, tgt),),
                    device_id_type=pl.DeviceIdType.MESH,
                )
                rdma.start()
                self.rdmas[s][d - 1] = rdma

    def finish_rs0(self):
        for s in range(NSCHED):
            for d in range(3):
                self.rdmas[s][d].wait()
            self.acc[pl.ds(self.q_off[s], QR), self._cols(s)] = (
                self.acc[pl.ds(self.q_off[s], QR), self._cols(s)]
                + self.rxq[s][self.b, 0].astype(jnp.float32)
                + self.rxq[s][self.b, 1].astype(jnp.float32)
                + self.rxq[s][self.b, 2].astype(jnp.float32))

    def start_rs1(self):
        for s in range(NSCHED):
            kind = SCHED_KINDS[s][1]
            for d in range(1, 4):
                tgt = (self.g1[s] + d) % 4
                off = pl.multiple_of(self.q_off[s] + tgt * SR, SR)
                self.sbs[s][self.b, d - 1, :, :] = self.acc[
                    pl.ds(off, SR), self._cols(s)].astype(jnp.bfloat16)
                rdma = pltpu.make_async_remote_copy(
                    src_ref=self.sbs[s].at[self.b, d - 1],
                    dst_ref=self.rxs[s].at[self.b, d - 1],
                    send_sem=self.rs1_s.at[self.b, s, d - 1],
                    recv_sem=self.rs1_r.at[self.b, s, d - 1],
                    device_id=(_member(self.me, kind, tgt),),
                    device_id_type=pl.DeviceIdType.MESH,
                )
                rdma.start()
                self.rdmas[s][d - 1] = rdma

    def finish_rs1(self):
        for s in range(NSCHED):
            for d in range(3):
                self.rdmas[s][d].wait()
            self.acc[pl.ds(self.f_off[s], SR), self._cols(s)] = (
                self.acc[pl.ds(self.f_off[s], SR), self._cols(s)]
                + self.rxs[s][self.b, 0].astype(jnp.float32)
                + self.rxs[s][self.b, 1].astype(jnp.float32)
                + self.rxs[s][self.b, 2].astype(jnp.float32))

    def start_ag0(self):
        for s in range(NSCHED):
            kind = SCHED_KINDS[s][1]
            self.agb[pl.ds(self.f_off[s], SR), self._cols(s)] = self.acc[
                pl.ds(self.f_off[s], SR), self._cols(s)].astype(jnp.bfloat16)
            for d in range(1, 4):
                tgt = (self.g1[s] + d) % 4
                rdma = pltpu.make_async_remote_copy(
                    src_ref=self.agb.at[pl.ds(self.f_off[s], SR),
                                        self._cols(s)],
                    dst_ref=self.agb.at[pl.ds(self.f_off[s], SR),
                                        self._cols(s)],
                    send_sem=self.ag0_s.at[self.b, s, d - 1],
                    recv_sem=self.ag0_r.at[self.b, s, d - 1],
                    device_id=(_member(self.me, kind, tgt),),
                    device_id_type=pl.DeviceIdType.MESH,
                )
                rdma.start()
                self.rdmas[s][d - 1] = rdma

    def finish_ag0(self):
        for s in range(NSCHED):
            for d in range(3):
                self.rdmas[s][d].wait()

    def start_ag1(self):
        for s in range(NSCHED):
            kind = SCHED_KINDS[s][0]
            for d in range(1, 4):
                tgt = (self.g0[s] + d) % 4
                rdma = pltpu.make_async_remote_copy(
                    src_ref=self.agb.at[pl.ds(self.q_off[s], QR),
                                        self._cols(s)],
                    dst_ref=self.agb.at[pl.ds(self.q_off[s], QR),
                                        self._cols(s)],
                    send_sem=self.ag1_s.at[self.b, s, d - 1],
                    recv_sem=self.ag1_r.at[self.b, s, d - 1],
                    device_id=(_member(self.me, kind, tgt),),
                    device_id_type=pl.DeviceIdType.MESH,
                )
                rdma.start()
                self.rdmas[s][d - 1] = rdma

    def finish_ag1(self):
        for s in range(NSCHED):
            for d in range(3):
                self.rdmas[s][d].wait()


def _compute_batch(b, x_ref, wq_ref, k_ref, v_ref, wo_ref, acc_ref, bias):
    HQ = SQ // 2
    bias_lo = bias[0:HQ, 0:HQ]
    bias_hi = bias[HQ:SQ, :]
    q = jnp.dot(x_ref[b], wq_ref[:, :],
                preferred_element_type=jnp.float32)
    ctx_parts = []
    for h in range(HL):
        qh = q[:, h * DH:(h + 1) * DH]
        kh = k_ref[b, :, h, :]
        vh = v_ref[b, :, h, :]
        s_lo = jnp.dot(qh[0:HQ, :], kh[0:HQ, :].T,
                       preferred_element_type=jnp.float32) * 0.125
        w_lo = jnp.exp(s_lo + bias_lo)
        ctx_lo = (jnp.dot(w_lo, vh[0:HQ, :],
                          preferred_element_type=jnp.float32)
                  / jnp.sum(w_lo, axis=-1, keepdims=True))
        s_hi = jnp.dot(qh[HQ:SQ, :], kh.T,
                       preferred_element_type=jnp.float32) * 0.125
        w_hi = jnp.exp(s_hi + bias_hi)
        ctx_hi = (jnp.dot(w_hi, vh,
                          preferred_element_type=jnp.float32)
                  / jnp.sum(w_hi, axis=-1, keepdims=True))
        ctx_parts.append(jnp.concatenate([ctx_lo, ctx_hi], axis=0))
    ctx = jnp.concatenate(ctx_parts, axis=1)
    acc_ref[b * SQ:(b + 1) * SQ, :] = jnp.dot(
        ctx, wo_ref[:, :], preferred_element_type=jnp.float32)


def _body(x_ref, wq_ref, k_ref, v_ref, wo_ref, out_ref,
          acc_ref, agb_ref, sbq0, sbq1, rxq0, rxq1, sbs0, sbs1, rxs0, rxs1,
          rs0_s, rs0_r, rs1_s, rs1_r, ag0_s, ag0_r, ag1_s, ag1_r):
    me = lax.axis_index("i")
    sbq, rxq = [sbq0, sbq1], [rxq0, rxq1]
    sbs, rxs = [sbs0, sbs1], [rxs0, rxs1]

    barrier = pltpu.get_barrier_semaphore()
    i = me & 3
    z = me >> 2
    for d in range(1, 4):
        pl.semaphore_signal(barrier, inc=1,
                            device_id=(_member(me, 0, (i + d) % 4),),
                            device_id_type=pl.DeviceIdType.MESH)
        pl.semaphore_signal(barrier, inc=1,
                            device_id=(_member(me, 1, (z + d) % 4),),
                            device_id_type=pl.DeviceIdType.MESH)
    pl.semaphore_wait(barrier, 6)

    qb = lax.broadcasted_iota(jnp.int32, (SQ, SQ), 0) // 64
    kb = lax.broadcasted_iota(jnp.int32, (SQ, SQ), 1) // 64
    bias = jnp.where(kb <= qb, 0.0, -1e9).astype(jnp.float32)

    sems = (rs0_s, rs0_r, rs1_s, rs1_r, ag0_s, ag0_r, ag1_s, ag1_r)
    ar0 = _BatchAllReduce(0, me, acc_ref, agb_ref, sbq, rxq, sbs, rxs, sems)
    ar1 = _BatchAllReduce(1, me, acc_ref, agb_ref, sbq, rxq, sbs, rxs, sems)

    _compute_batch(0, x_ref, wq_ref, k_ref, v_ref, wo_ref, acc_ref, bias)
    ar0.start_rs0()
    _compute_batch(1, x_ref, wq_ref, k_ref, v_ref, wo_ref, acc_ref, bias)

    ar0.finish_rs0(); ar0.start_rs1()
    ar1.start_rs0()
    ar0.finish_rs1(); ar0.start_ag0()
    ar1.finish_rs0(); ar1.start_rs1()
    ar0.finish_ag0(); ar0.start_ag1()
    ar1.finish_rs1(); ar1.start_ag0()
    ar0.finish_ag1()
    out_ref[0] = agb_ref[0:SQ, :].astype(jnp.float32)
    ar1.finish_ag0(); ar1.start_ag1()
    ar1.finish_ag1()
    out_ref[1] = agb_ref[SQ:ROWS, :].astype(jnp.float32)


def kernel(x, Wq, K_ext, V_ext, Wo):
    me = lax.axis_index("i")
    wq_loc = lax.dynamic_slice(Wq, (0, me * D_LOC), (Wq.shape[0], D_LOC))
    wo_loc = lax.dynamic_slice(Wo, (me * D_LOC, 0), (D_LOC, Wo.shape[1]))

    return pl.pallas_call(
        _body,
        out_shape=jax.ShapeDtypeStruct((B, SQ, D_MODEL), jnp.float32),
        in_specs=[pl.BlockSpec(memory_space=pltpu.VMEM)] * 5,
        out_specs=pl.BlockSpec(memory_space=pltpu.VMEM),
        scratch_shapes=[
            pltpu.VMEM((ROWS, D_MODEL), jnp.float32),
            pltpu.VMEM((ROWS, D_MODEL), jnp.bfloat16),
            pltpu.VMEM((B, 3, QR, COLS[0][1]), jnp.bfloat16),
            pltpu.VMEM((B, 3, QR, COLS[1][1]), jnp.bfloat16),
            pltpu.VMEM((B, 3, QR, COLS[0][1]), jnp.bfloat16),
            pltpu.VMEM((B, 3, QR, COLS[1][1]), jnp.bfloat16),
            pltpu.VMEM((B, 3, SR, COLS[0][1]), jnp.bfloat16),
            pltpu.VMEM((B, 3, SR, COLS[1][1]), jnp.bfloat16),
            pltpu.VMEM((B, 3, SR, COLS[0][1]), jnp.bfloat16),
            pltpu.VMEM((B, 3, SR, COLS[1][1]), jnp.bfloat16),
            pltpu.SemaphoreType.DMA((B, NSCHED, 3)),
            pltpu.SemaphoreType.DMA((B, NSCHED, 3)),
            pltpu.SemaphoreType.DMA((B, NSCHED, 3)),
            pltpu.SemaphoreType.DMA((B, NSCHED, 3)),
            pltpu.SemaphoreType.DMA((B, NSCHED, 3)),
            pltpu.SemaphoreType.DMA((B, NSCHED, 3)),
            pltpu.SemaphoreType.DMA((B, NSCHED, 3)),
            pltpu.SemaphoreType.DMA((B, NSCHED, 3)),
        ],
        compiler_params=pltpu.CompilerParams(collective_id=0),
    )(x, wq_loc, K_ext, V_ext, wo_loc)
